# baseline (speedup 1.0000x reference)
import functools

import jax
import jax.numpy as jnp
from jax import lax
from jax.experimental import pallas as pl
from jax.experimental.pallas import tpu as pltpu
from jax.experimental.pallas import tpu_sc as plsc

_BATCH = 16384
_HIST = 200
_EMBED = 32
_B = _BATCH * _HIST

_NC = 2
_NS = 16
_NW = _NC * _NS

_B_PER_W = _B // _NW
_CHUNK = 1600
_N_CHUNKS = _B_PER_W // _CHUNK
_N_GROUPS = _N_CHUNKS // 2


def _embed_kernel(idx_hbm, table_hbm, out_hbm, idx_v, rows_v,
                  si0, si1, sg0, sg1, ss0, ss1):
    wid = lax.axis_index("s") * _NC + lax.axis_index("c")
    base = wid * _B_PER_W
    si = (si0, si1)
    sg = (sg0, sg1)
    ss = (ss0, ss1)

    def idx_cp(g, b):
        return pltpu.make_async_copy(
            idx_hbm.at[pl.ds(base + g * _CHUNK, _CHUNK)], idx_v.at[b], si[b])

    def gat_cp(b):
        return pltpu.make_async_copy(
            table_hbm.at[idx_v.at[b]], rows_v.at[b], sg[b])

    def sct_cp(g, b):
        return pltpu.make_async_copy(
            rows_v.at[b], out_hbm.at[pl.ds(base + g * _CHUNK, _CHUNK)], ss[b])

    for b in range(2):
        idx_cp(b, b).start()
    for b in range(2):
        idx_cp(b, b).wait()
        gat_cp(b).start()
    for b in range(2):
        gat_cp(b).wait()
        sct_cp(b, b).start()
        idx_cp(2 + b, b).start()

    def body(i, carry):
        g0 = 2 * i
        for b in range(2):
            idx_cp(g0 + b, b).wait()
            sct_cp(g0 + b, b).wait()
            gat_cp(b).start()
        for b in range(2):
            gat_cp(b).wait()
            sct_cp(g0 + b, b).start()
            idx_cp(g0 + 2 + b, b).start()
        return carry

    lax.fori_loop(1, _N_GROUPS - 1, body, 0)

    g0 = 2 * (_N_GROUPS - 1)
    for b in range(2):
        idx_cp(g0 + b, b).wait()
        sct_cp(g0 + b, b).wait()
        gat_cp(b).start()
    for b in range(2):
        gat_cp(b).wait()
        sct_cp(g0 + b, b).start()
    for b in range(2):
        sct_cp(g0 + b, b).wait()


@jax.jit
def _embed(x_flat, table):
    mesh = plsc.VectorSubcoreMesh(core_axis_name="c", subcore_axis_name="s")
    k = functools.partial(
        pl.kernel,
        mesh=mesh,
        out_type=jax.ShapeDtypeStruct((_B, _EMBED), jnp.float32),
        scratch_types=[
            pltpu.VMEM((2, _CHUNK), jnp.int32),
            pltpu.VMEM((2, _CHUNK, _EMBED), jnp.float32),
            pltpu.SemaphoreType.DMA,
            pltpu.SemaphoreType.DMA,
            pltpu.SemaphoreType.DMA,
            pltpu.SemaphoreType.DMA,
            pltpu.SemaphoreType.DMA,
            pltpu.SemaphoreType.DMA,
        ],
        compiler_params=pltpu.CompilerParams(use_tc_tiling_on_sc=False),
    )(_embed_kernel)
    return k(x_flat, table)


def kernel(x, table):
    x_flat = x.reshape(-1).astype(jnp.int32)
    out = _embed(x_flat, table)
    return out.reshape(_BATCH, _HIST, _EMBED)
